# baseline (speedup 1.0000x reference)
import functools

import jax
import jax.numpy as jnp
from jax import lax
from jax.experimental import pallas as pl
from jax.experimental.pallas import tpu as pltpu
from jax.experimental.pallas import tpu_sc as plsc

N = 16384
NUM_TILES = 16
CHUNK = N // NUM_TILES
LANES = 16
ROW = 16
HIST = 7 * ROW

_mesh = plsc.VectorSubcoreMesh(core_axis_name="c", subcore_axis_name="s",
                               num_cores=1)


@functools.partial(
    pl.kernel,
    mesh=_mesh,
    compiler_params=pltpu.CompilerParams(needs_layout_passes=False),
    out_type=jax.ShapeDtypeStruct((1,), jnp.float32),
    scratch_types=[
        pltpu.VMEM((CHUNK,), jnp.int32),
        pltpu.VMEM((CHUNK,), jnp.int32),
        pltpu.VMEM((HIST,), jnp.float32),
        pltpu.VMEM_SHARED((NUM_TILES, HIST), jnp.float32),
        pltpu.VMEM((NUM_TILES, HIST), jnp.float32),
        pltpu.VMEM((LANES,), jnp.float32),
    ],
)
def _f1_sc(pred_hbm, real_hbm, out_hbm, pred_v, real_v, hist_v, shared,
           all_v, out_v):
    cid = lax.axis_index("c")
    sid = lax.axis_index("s")
    lane = lax.iota(jnp.int32, LANES)
    zero16 = jnp.zeros((LANES,), jnp.float32)
    ones16 = jnp.ones((LANES,), jnp.float32)

    @pl.when(cid == 0)
    def _histogram():
        base = sid * CHUNK
        pltpu.sync_copy(pred_hbm.at[pl.ds(base, CHUNK)], pred_v)
        pltpu.sync_copy(real_hbm.at[pl.ds(base, CHUNK)], real_v)
        for i in range(HIST // LANES):
            hist_v[pl.ds(i * LANES, LANES)] = zero16
        for i in range(CHUNK // LANES):
            p = pred_v[pl.ds(i * LANES, LANES)]
            r = real_v[pl.ds(i * LANES, LANES)]
            bins = r * ROW + p
            plsc.addupdate_scatter(hist_v, [bins], ones16)
        pltpu.sync_copy(hist_v, shared.at[sid])

    plsc.subcore_barrier()

    @pl.when(jnp.logical_and(cid == 0, sid == 0))
    def _reduce_and_f1():
        pltpu.sync_copy(shared, all_v)
        rows = []
        for r in range(7):
            acc = zero16
            for t in range(NUM_TILES):
                acc = acc + all_v[t, pl.ds(r * ROW, LANES)]
            rows.append(acc)

        n2v = zero16
        diagv = zero16
        nv = zero16
        for r in range(7):
            n2v = n2v + rows[r]
            rmask = lane == r
            diagv = diagv + jnp.where(rmask, rows[r], 0.0)
            nv = nv + jnp.where(rmask, jnp.sum(rows[r]), 0.0)

        totalv = zero16 + jnp.sum(nv)
        weight = nv / totalv
        recall = diagv / jnp.maximum(nv, 1.0)
        precision = diagv / jnp.maximum(n2v, 1.0)
        denom = precision + recall
        f1 = jnp.where(denom > 0.0,
                       2.0 * precision * recall / jnp.maximum(denom, 1e-30),
                       0.0)
        f1sum_v = zero16 + jnp.sum(f1)
        wsum_v = zero16 + jnp.sum(weight)
        seven_v = zero16 + 7.0
        loss_v = 1.0 - (f1sum_v / seven_v) * wsum_v

        empty_cols = jnp.logical_and(n2v == 0.0, lane < 7)
        n_empty = plsc.all_reduce_population_count(empty_cols)
        first_nz = plsc.all_reduce_ffs(n2v != 0.0)
        w_idx_v = zero16 + jnp.sum(jnp.where(lane == first_nz, weight, 0.0))
        loss_v = jnp.where(n_empty > 5, loss_v + loss_v * w_idx_v * 100.0,
                           loss_v)
        out_v[...] = loss_v
        pltpu.sync_copy(out_v.at[pl.ds(0, 1)], out_hbm)


def kernel(pred, real):
    out = _f1_sc(pred.astype(jnp.int32), real.astype(jnp.int32))
    return jnp.reshape(out, ())


# speedup vs baseline: 1.0203x; 1.0203x over previous
import functools

import jax
import jax.numpy as jnp
from jax import lax
from jax.experimental import pallas as pl
from jax.experimental.pallas import tpu as pltpu
from jax.experimental.pallas import tpu_sc as plsc

N = 16384
NUM_TILES = 16
CHUNK = N // NUM_TILES
LANES = 16
ROWS = 16

_mesh = plsc.VectorSubcoreMesh(core_axis_name="c", subcore_axis_name="s",
                               num_cores=1)


@functools.partial(
    pl.kernel,
    mesh=_mesh,
    compiler_params=pltpu.CompilerParams(needs_layout_passes=False),
    out_type=jax.ShapeDtypeStruct((1,), jnp.float32),
    scratch_types=[
        pltpu.VMEM((CHUNK,), jnp.int32),
        pltpu.VMEM((CHUNK,), jnp.int32),
        pltpu.VMEM((ROWS * LANES,), jnp.float32),
        pltpu.VMEM_SHARED((NUM_TILES, ROWS * LANES), jnp.float32),
        pltpu.VMEM((NUM_TILES, ROWS * LANES), jnp.float32),
        pltpu.VMEM((LANES,), jnp.float32),
        pltpu.SemaphoreType.DMA,
        pltpu.SemaphoreType.DMA,
    ],
)
def _f1_sc(pred_hbm, real_hbm, out_hbm, pred_v, real_v, hist_v, shared,
           all_v, out_v, sem_p, sem_r):
    cid = lax.axis_index("c")
    sid = lax.axis_index("s")
    lane = lax.iota(jnp.int32, LANES)
    zero16 = jnp.zeros((LANES,), jnp.float32)
    ones16 = jnp.ones((LANES,), jnp.float32)

    base = sid * CHUNK
    cp_p = pltpu.async_copy(pred_hbm.at[pl.ds(base, CHUNK)], pred_v, sem_p)
    cp_r = pltpu.async_copy(real_hbm.at[pl.ds(base, CHUNK)], real_v, sem_r)
    for r in range(ROWS):
        hist_v[pl.ds(r * LANES, LANES)] = zero16
    cp_p.wait()
    cp_r.wait()
    for i in range(CHUNK // LANES):
        p = pred_v[pl.ds(i * LANES, LANES)]
        r = real_v[pl.ds(i * LANES, LANES)]
        plsc.addupdate_scatter(hist_v, [r * LANES + p], ones16)

    pltpu.sync_copy(hist_v, shared.at[sid])
    plsc.subcore_barrier()

    @pl.when(sid == 0)
    def _reduce_and_f1():
        pltpu.sync_copy(shared, all_v)
        rows = []
        for r in range(7):
            acc = zero16
            for t in range(NUM_TILES):
                acc = acc + all_v[t, pl.ds(r * LANES, LANES)]
            rows.append(acc)

        n2v = zero16
        diagv = zero16
        nv = zero16
        for r in range(7):
            n2v = n2v + rows[r]
            rmask = lane == r
            diagv = diagv + jnp.where(rmask, rows[r], 0.0)
            nv = nv + jnp.where(rmask, jnp.sum(rows[r]), 0.0)

        totalv = zero16 + jnp.sum(nv)
        weight = nv / totalv
        recall = diagv / jnp.maximum(nv, 1.0)
        precision = diagv / jnp.maximum(n2v, 1.0)
        denom = precision + recall
        f1 = jnp.where(denom > 0.0,
                       2.0 * precision * recall / jnp.maximum(denom, 1e-30),
                       0.0)
        f1sum_v = zero16 + jnp.sum(f1)
        wsum_v = zero16 + jnp.sum(weight)
        seven_v = zero16 + 7.0
        loss_v = 1.0 - (f1sum_v / seven_v) * wsum_v

        empty_cols = jnp.logical_and(n2v == 0.0, lane < 7)
        n_empty = plsc.all_reduce_population_count(empty_cols)
        first_nz = plsc.all_reduce_ffs(n2v != 0.0)
        w_idx_v = zero16 + jnp.sum(jnp.where(lane == first_nz, weight, 0.0))
        loss_v = jnp.where(n_empty > 5, loss_v + loss_v * w_idx_v * 100.0,
                           loss_v)
        out_v[...] = loss_v
        pltpu.sync_copy(out_v.at[pl.ds(0, 1)], out_hbm)


def kernel(pred, real):
    out = _f1_sc(pred.astype(jnp.int32), real.astype(jnp.int32))
    return jnp.reshape(out, ())


# speedup vs baseline: 1.0795x; 1.0579x over previous
import functools

import jax
import jax.numpy as jnp
from jax import lax
from jax.experimental import pallas as pl
from jax.experimental.pallas import tpu as pltpu
from jax.experimental.pallas import tpu_sc as plsc

N = 16384
NUM_TILES = 16
CHUNK = N // NUM_TILES
LANES = 16
ROWS = 16

_mesh = plsc.VectorSubcoreMesh(core_axis_name="c", subcore_axis_name="s",
                               num_cores=1)


@functools.partial(
    pl.kernel,
    mesh=_mesh,
    compiler_params=pltpu.CompilerParams(needs_layout_passes=False),
    out_type=jax.ShapeDtypeStruct((1,), jnp.float32),
    scratch_types=[
        pltpu.VMEM((CHUNK,), jnp.int32),
        pltpu.VMEM((CHUNK,), jnp.int32),
        pltpu.VMEM((ROWS * LANES,), jnp.float32),
        pltpu.VMEM_SHARED((NUM_TILES, 7 * LANES), jnp.float32),
        pltpu.VMEM((NUM_TILES, 7 * LANES), jnp.float32),
        pltpu.VMEM((LANES,), jnp.float32),
        pltpu.SemaphoreType.DMA,
        pltpu.SemaphoreType.DMA,
    ],
)
def _f1_sc(pred_hbm, real_hbm, out_hbm, pred_v, real_v, hist_v, shared,
           all_v, out_v, sem_p, sem_r):
    cid = lax.axis_index("c")
    sid = lax.axis_index("s")
    lane = lax.iota(jnp.int32, LANES)
    zero16 = jnp.zeros((LANES,), jnp.float32)
    ones16 = jnp.ones((LANES,), jnp.float32)

    base = sid * CHUNK
    cp_p = pltpu.async_copy(pred_hbm.at[pl.ds(base, CHUNK)], pred_v, sem_p)
    cp_r = pltpu.async_copy(real_hbm.at[pl.ds(base, CHUNK)], real_v, sem_r)
    for r in range(ROWS):
        hist_v[pl.ds(r * LANES, LANES)] = zero16
    cp_p.wait()
    cp_r.wait()
    @plsc.parallel_loop(0, CHUNK, step=LANES, unroll=4)
    def _scatter(i):
        p = pred_v[pl.ds(i, LANES)]
        r = real_v[pl.ds(i, LANES)]
        plsc.addupdate_scatter(hist_v, [r * LANES + p], ones16)

    pltpu.sync_copy(hist_v.at[pl.ds(0, 7 * LANES)], shared.at[sid])
    plsc.subcore_barrier()

    @pl.when(sid == 0)
    def _reduce_and_f1():
        pltpu.sync_copy(shared, all_v)
        rows = []
        for r in range(7):
            acc = zero16
            for t in range(NUM_TILES):
                acc = acc + all_v[t, pl.ds(r * LANES, LANES)]
            rows.append(acc)

        n2v = zero16
        diagv = zero16
        nv = zero16
        for r in range(7):
            n2v = n2v + rows[r]
            rmask = lane == r
            diagv = diagv + jnp.where(rmask, rows[r], 0.0)
            nv = nv + jnp.where(rmask, jnp.sum(rows[r]), 0.0)

        totalv = zero16 + jnp.sum(nv)
        weight = nv / totalv
        recall = diagv / jnp.maximum(nv, 1.0)
        precision = diagv / jnp.maximum(n2v, 1.0)
        denom = precision + recall
        f1 = jnp.where(denom > 0.0,
                       2.0 * precision * recall / jnp.maximum(denom, 1e-30),
                       0.0)
        f1sum_v = zero16 + jnp.sum(f1)
        wsum_v = zero16 + jnp.sum(weight)
        seven_v = zero16 + 7.0
        loss_v = 1.0 - (f1sum_v / seven_v) * wsum_v

        empty_cols = jnp.logical_and(n2v == 0.0, lane < 7)
        n_empty = plsc.all_reduce_population_count(empty_cols)
        first_nz = plsc.all_reduce_ffs(n2v != 0.0)
        w_idx_v = zero16 + jnp.sum(jnp.where(lane == first_nz, weight, 0.0))
        loss_v = jnp.where(n_empty > 5, loss_v + loss_v * w_idx_v * 100.0,
                           loss_v)
        out_v[...] = loss_v
        pltpu.sync_copy(out_v.at[pl.ds(0, 1)], out_hbm)


def kernel(pred, real):
    out = _f1_sc(pred.astype(jnp.int32), real.astype(jnp.int32))
    return jnp.reshape(out, ())
